# baseline (speedup 1.0000x reference)
import functools

import jax
import jax.numpy as jnp
from jax import lax
from jax.experimental import pallas as pl
from jax.experimental.pallas import tpu as pltpu
from jax.experimental.pallas import tpu_sc as plsc

_NC = 2
_NS = 16
_NW = _NC * _NS

_f32 = jnp.float32


def _leaky(x):
    return jnp.where(x > 0, x, 0.01 * x)


def _dot(a, b):
    return lax.dot(a, b, preferred_element_type=_f32)


def _prep_body(vis_ref, wproj_ref, bproj_ref, wps_ref, bps_ref, wl_ref,
               bl_ref, wr_ref, br_ref, qts_ref, qrel_ref,
               hn_ref, lq_ref, rq_ref):
    wproj = wproj_ref[...]
    bproj = bproj_ref[...]
    wl = wl_ref[...]
    wr = wr_ref[...]
    ds = wproj.shape[1]
    sds = wps_ref.shape[1]
    wl_c, wl_d = wl[2 * ds:2 * ds + sds], wl[2 * ds + sds:]
    wr_c, wr_d = wr[2 * ds:2 * ds + sds], wr[2 * ds + sds:]
    hn_ref[...] = _dot(vis_ref[...], wproj) + bproj
    qsv = _dot(qts_ref[...], wps_ref[...]) + bps_ref[...]
    qrv = _dot(qrel_ref[...], wproj) + bproj
    lq_ref[...] = _dot(qsv, wl_c) + _dot(qrv, wl_d) + bl_ref[...]
    rq_ref[...] = _dot(qsv, wr_c) + _dot(qrv, wr_d) + br_ref[...]


def _prep_call(N, B, D, DS, SDS):
    return pl.pallas_call(
        _prep_body,
        out_shape=(
            jax.ShapeDtypeStruct((N, DS), _f32),
            jax.ShapeDtypeStruct((B, DS), _f32),
            jax.ShapeDtypeStruct((B, DS), _f32),
        ),
    )


def _sc_gather_call(N, E, DS):
    R = E // 2
    RW = R // _NW
    SUP = 1000
    CH = 40
    NSUP = RW // SUP
    NCH = SUP // CH
    assert RW % SUP == 0 and SUP % CH == 0 and RW % 8 == 0

    mesh = plsc.VectorSubcoreMesh(core_axis_name="c", subcore_axis_name="s")

    @functools.partial(
        pl.kernel,
        out_type=(
            jax.ShapeDtypeStruct((R, 4 * DS), _f32),
            jax.ShapeDtypeStruct((R,), _f32),
            jax.ShapeDtypeStruct((R,), _f32),
        ),
        mesh=mesh,
        scratch_types=[
            pltpu.VMEM((RW,), jnp.int32),
            pltpu.VMEM((SUP, DS), _f32),
            pltpu.VMEM((RW,), _f32),
            pltpu.SemaphoreType.DMA,
        ],
        compiler_params=pltpu.CompilerParams(use_tc_tiling_on_sc=False),
    )
    def sc_gather(hn_hbm, ns_hbm, iie_hbm, jje_hbm, iio_hbm, jjo_hbm,
                  hnp_hbm, nse_hbm, nso_hbm,
                  idx_v, rows_v, nsout_v, gsem):
        cid = lax.axis_index("c")
        sid = lax.axis_index("s")
        wid = sid * _NC + cid
        base = wid * RW

        def do_quarter(idx_hbm, col, ns_out_hbm):
            pltpu.sync_copy(idx_hbm.at[pl.ds(base, RW)], idx_v)
            for s in range(NSUP):
                off = s * SUP

                def fire(cc, carry):
                    pltpu.async_copy(
                        hn_hbm.at[idx_v.at[pl.ds(off + cc * CH, CH)]],
                        rows_v.at[pl.ds(cc * CH, CH)], gsem)
                    if ns_out_hbm is not None:
                        pltpu.async_copy(
                            ns_hbm.at[idx_v.at[pl.ds(off + cc * CH, CH)]],
                            nsout_v.at[pl.ds(off + cc * CH, CH)], gsem)
                    return carry

                def drain(cc, carry):
                    pltpu.make_async_copy(
                        hn_hbm.at[idx_v.at[pl.ds(off + cc * CH, CH)]],
                        rows_v.at[pl.ds(cc * CH, CH)], gsem).wait()
                    if ns_out_hbm is not None:
                        pltpu.make_async_copy(
                            ns_hbm.at[idx_v.at[pl.ds(off + cc * CH, CH)]],
                            nsout_v.at[pl.ds(off + cc * CH, CH)], gsem).wait()
                    return carry

                lax.fori_loop(0, NCH, fire, 0)
                lax.fori_loop(0, NCH, drain, 0)
                pltpu.sync_copy(
                    rows_v,
                    hnp_hbm.at[pl.ds(base + off, SUP), pl.ds(col, DS)])
            if ns_out_hbm is not None:
                pltpu.sync_copy(nsout_v, ns_out_hbm.at[pl.ds(base, RW)])

        do_quarter(iie_hbm, 0, nse_hbm)
        do_quarter(jje_hbm, DS, None)
        do_quarter(iio_hbm, 2 * DS, nso_hbm)
        do_quarter(jjo_hbm, 3 * DS, None)

    return sc_gather


def _edge_body(rel_ref, hnp_ref, lq_ref, rq_ref,
               wproj_ref, bproj_ref, wl_ref, wr_ref, wc_ref, bc_ref,
               logit_ref):
    ds = wproj_ref.shape[1]
    d = wproj_ref.shape[0]
    wl = wl_ref[...]
    wr = wr_ref[...]
    wc = wc_ref[...]
    z = jnp.zeros((ds, ds), _f32)
    zd = jnp.zeros((d, ds), _f32)
    z64 = jnp.zeros((2 * ds, 2 * ds), _f32)
    wd64 = jnp.concatenate(
        [jnp.concatenate([wl[0:ds], z], axis=1),
         jnp.concatenate([z, wr[0:ds]], axis=1)], axis=0)
    wd128 = jnp.concatenate(
        [jnp.concatenate([wd64, z64], axis=1),
         jnp.concatenate([z64, wd64], axis=1)], axis=0)
    wproj2 = jnp.concatenate(
        [jnp.concatenate([wproj_ref[...], zd], axis=1),
         jnp.concatenate([zd, wproj_ref[...]], axis=1)], axis=0)
    wrelb = jnp.concatenate([wl[ds:2 * ds], wr[ds:2 * ds]], axis=1)
    zrb = jnp.zeros_like(wrelb)
    wrelb2 = jnp.concatenate(
        [jnp.concatenate([wrelb, zrb], axis=1),
         jnp.concatenate([zrb, wrelb], axis=1)], axis=0)
    wc2p = jnp.concatenate(
        [jnp.zeros((ds, 2 * ds), _f32),
         jnp.concatenate([wc, z], axis=1)], axis=0)
    wcp = jnp.concatenate(
        [jnp.concatenate([wc2p, z64], axis=1),
         jnp.concatenate([z64, wc2p], axis=1)], axis=0)
    bproj = bproj_ref[...]
    bproj2 = jnp.concatenate([bproj, bproj], axis=1)
    bc = bc_ref[...]
    zb = jnp.zeros_like(bc)
    bcp = jnp.concatenate([bc, zb, bc, zb], axis=1)
    lqrq = jnp.concatenate(
        [lq_ref[0], rq_ref[0], lq_ref[0], rq_ref[0]], axis=1)

    relp2 = _dot(rel_ref[0], wproj2) + bproj2
    alar = _leaky(_dot(hnp_ref[0], wd128) + _dot(relp2, wrelb2) + lqrq)
    m2 = _dot(alar, wcp) + bcp
    prod = alar * m2
    le = jnp.sum(prod[:, 0:2 * ds], axis=1, keepdims=True)
    lo = jnp.sum(prod[:, 2 * ds:4 * ds], axis=1, keepdims=True)
    logit_ref[0] = jnp.concatenate([le, lo], axis=1)


def _edge_call(B, EQ, D, DS, GIN):
    H = EQ // 2
    return pl.pallas_call(
        _edge_body,
        grid=(B,),
        in_specs=[
            pl.BlockSpec((1, H, 2 * D), lambda b: (b, 0, 0)),
            pl.BlockSpec((1, H, 4 * DS), lambda b: (b, 0, 0)),
            pl.BlockSpec((1, 1, DS), lambda b: (b, 0, 0)),
            pl.BlockSpec((1, 1, DS), lambda b: (b, 0, 0)),
            pl.BlockSpec((D, DS), lambda b: (0, 0)),
            pl.BlockSpec((1, DS), lambda b: (0, 0)),
            pl.BlockSpec((GIN, DS), lambda b: (0, 0)),
            pl.BlockSpec((GIN, DS), lambda b: (0, 0)),
            pl.BlockSpec((DS, DS), lambda b: (0, 0)),
            pl.BlockSpec((1, DS), lambda b: (0, 0)),
        ],
        out_specs=pl.BlockSpec((1, H, 2), lambda b: (b, 0, 0)),
        out_shape=jax.ShapeDtypeStruct((B, H, 2), _f32),
    )


def _topk_body(k_ref, lg_ref, ns_ref, out_ref):
    lg = lg_ref[...]
    m = jnp.max(lg, axis=1, keepdims=True)
    p = jnp.exp(lg - m)
    z = jnp.sum(p, axis=1, keepdims=True)
    att = (p / z) * ns_ref[...]
    bits = lax.bitcast_convert_type(att, jnp.int32)
    k = k_ref[0]
    b = att.shape[0]

    def body(_, carry):
        lo, hi = carry
        mid = lo + lax.shift_right_logical(hi - lo, 1)
        cnt = jnp.sum((bits >= mid).astype(jnp.int32), axis=1, keepdims=True)
        ge = cnt >= k
        return jnp.where(ge, mid, lo), jnp.where(ge, hi, mid)

    lo0 = jnp.zeros((b, 1), jnp.int32)
    hi0 = jnp.full((b, 1), jnp.int32(0x7FFFFFFF))
    lo, _ = lax.fori_loop(0, 31, body, (lo0, hi0))
    out_ref[...] = jnp.where(bits >= lo, att, 0.0)


def _topk_call(B, EQ):
    return pl.pallas_call(
        _topk_body,
        in_specs=[
            pl.BlockSpec(memory_space=pltpu.SMEM),
            pl.BlockSpec((B, EQ), lambda: (0, 0)),
            pl.BlockSpec((B, EQ), lambda: (0, 0)),
        ],
        out_specs=pl.BlockSpec((B, EQ), lambda: (0, 0)),
        out_shape=jax.ShapeDtypeStruct((B, EQ), _f32),
    )


def _sc_scatter_call(N, JC):
    mesh = plsc.VectorSubcoreMesh(core_axis_name="c", subcore_axis_name="s")

    @functools.partial(
        pl.kernel,
        out_type=jax.ShapeDtypeStruct((_NC, N), _f32),
        mesh=mesh,
        scratch_types=[
            pltpu.VMEM((JC, 128), _f32),
            pltpu.VMEM((JC, 128), jnp.int32),
            pltpu.VMEM((N,), _f32),
            pltpu.VMEM_SHARED((N,), _f32),
            pltpu.SemaphoreType.DMA,
        ],
    )
    def sc_scatter(val_hbm, idx_hbm, out_hbm, val_v, idx_v, z_v, acc_sh, sem):
        cid = lax.axis_index("c")
        sid = lax.axis_index("s")
        wid = sid * _NC + cid

        @pl.when(sid == 0)
        def _zero():
            def zb(t, carry):
                z_v[pl.ds(t * 16, 16)] = jnp.zeros((16,), _f32)
                return carry
            lax.fori_loop(0, N // 16, zb, 0)
            pltpu.sync_copy(z_v, acc_sh)

        pltpu.sync_copy(val_hbm.at[wid], val_v)
        pltpu.sync_copy(idx_hbm.at[wid], idx_v)
        plsc.subcore_barrier()

        def scat(j, carry):
            pltpu.sync_copy(val_v.at[j], acc_sh.at[idx_v.at[j]], add=True)
            return carry
        lax.fori_loop(0, JC, scat, 0)

        plsc.subcore_barrier()

        @pl.when(sid == 0)
        def _out():
            pltpu.sync_copy(acc_sh, out_hbm.at[cid])

    return sc_scatter


def _add_body(p_ref, o_ref):
    o_ref[...] = p_ref[0, :] + p_ref[1, :]


def _add_call(N):
    return pl.pallas_call(
        _add_body,
        out_shape=jax.ShapeDtypeStruct((N,), _f32),
    )


def kernel(node_score, selected_edges_l, visited_node_representation,
           rel_emb_l, query_src_ts_emb, query_rel_emb, W_proj, b_proj,
           W_ps, b_ps, Wl, bl, Wr, br, Wc, bc, max_edges):
    B = query_rel_emb.shape[0]
    E = selected_edges_l.shape[0]
    EQ = E // B
    N = node_score.shape[0]
    D = visited_node_representation.shape[1]
    DS = W_proj.shape[1]
    SDS = W_ps.shape[1]

    GIN = Wl.shape[0]

    hn, lq, rq = _prep_call(N, B, D, DS, SDS)(
        visited_node_representation, W_proj, b_proj.reshape(1, DS),
        W_ps, b_ps.reshape(1, SDS), Wl, bl.reshape(1, DS),
        Wr, br.reshape(1, DS), query_src_ts_emb, query_rel_emb)

    idx_j = selected_edges_l[:, 7]
    iie = selected_edges_l[0::2, 6]
    iio = selected_edges_l[1::2, 6]
    jje = selected_edges_l[0::2, 7]
    jjo = selected_edges_l[1::2, 7]

    hnp, nse, nso = _sc_gather_call(N, E, DS)(
        hn, node_score, iie, jje, iio, jjo)

    logits = _edge_call(B, EQ, D, DS, GIN)(
        rel_emb_l.reshape(B, EQ // 2, 2 * D),
        hnp.reshape(B, EQ // 2, 4 * DS),
        lq.reshape(B, 1, DS), rq.reshape(B, 1, DS), W_proj,
        b_proj.reshape(1, DS), Wl, Wr, Wc, bc.reshape(1, DS))

    nsi = jnp.stack([nse, nso], axis=1)

    pruned = _topk_call(B, EQ)(
        jnp.asarray(max_edges, jnp.int32).reshape(1),
        logits.reshape(B, EQ), nsi.reshape(B, EQ))

    JC = -(-(E // _NW) // 128)
    EPAD = _NW * JC * 128
    pv = jnp.concatenate([pruned.reshape(E),
                          jnp.zeros((EPAD - E,), _f32)])
    pj = jnp.concatenate([idx_j, jnp.zeros((EPAD - E,), jnp.int32)])

    partials = _sc_scatter_call(N, JC)(
        pv.reshape(_NW, JC, 128), pj.reshape(_NW, JC, 128))

    return _add_call(N)(partials)


# speedup vs baseline: 1.8198x; 1.8198x over previous
import functools

import jax
import jax.numpy as jnp
from jax import lax
from jax.experimental import pallas as pl
from jax.experimental.pallas import tpu as pltpu
from jax.experimental.pallas import tpu_sc as plsc

_NC = 2
_NS = 16
_NW = _NC * _NS

_f32 = jnp.float32


def _leaky(x):
    return jnp.where(x > 0, x, 0.01 * x)


def _dot(a, b):
    return lax.dot(a, b, preferred_element_type=_f32)


def _prep_body(vis_ref, wproj_ref, bproj_ref, wps_ref, bps_ref, wl_ref,
               bl_ref, wr_ref, br_ref, qts_ref, qrel_ref,
               hn_ref, lq_ref, rq_ref):
    wproj = wproj_ref[...]
    bproj = bproj_ref[...]
    wl = wl_ref[...]
    wr = wr_ref[...]
    ds = wproj.shape[1]
    sds = wps_ref.shape[1]
    wl_c, wl_d = wl[2 * ds:2 * ds + sds], wl[2 * ds + sds:]
    wr_c, wr_d = wr[2 * ds:2 * ds + sds], wr[2 * ds + sds:]
    hn_ref[...] = _dot(vis_ref[...], wproj) + bproj
    qsv = _dot(qts_ref[...], wps_ref[...]) + bps_ref[...]
    qrv = _dot(qrel_ref[...], wproj) + bproj
    lq_ref[...] = _dot(qsv, wl_c) + _dot(qrv, wl_d) + bl_ref[...]
    rq_ref[...] = _dot(qsv, wr_c) + _dot(qrv, wr_d) + br_ref[...]


def _prep_call(N, B, D, DS, SDS):
    return pl.pallas_call(
        _prep_body,
        out_shape=(
            jax.ShapeDtypeStruct((N, DS), _f32),
            jax.ShapeDtypeStruct((B, DS), _f32),
            jax.ShapeDtypeStruct((B, DS), _f32),
        ),
    )


def _sc_gather_call(N, E, DS):
    EW = E // _NW
    SUP = 2000
    CH = 80
    NSUP = EW // SUP
    NCH = SUP // CH
    assert EW % SUP == 0 and SUP % CH == 0 and EW % 16 == 0

    mesh = plsc.VectorSubcoreMesh(core_axis_name="c", subcore_axis_name="s")

    @functools.partial(
        pl.kernel,
        out_type=(
            jax.ShapeDtypeStruct((E, 2 * DS), _f32),
            jax.ShapeDtypeStruct((E,), _f32),
        ),
        mesh=mesh,
        scratch_types=[
            pltpu.VMEM((EW,), jnp.int32),
            pltpu.VMEM((SUP, DS), _f32),
            pltpu.VMEM((EW,), _f32),
            pltpu.SemaphoreType.DMA,
        ],
        compiler_params=pltpu.CompilerParams(use_tc_tiling_on_sc=False),
    )
    def sc_gather(hn_hbm, ns_hbm, ii_hbm, jj_hbm,
                  hnij_hbm, nsi_hbm,
                  idx_v, rows_v, nsout_v, gsem):
        cid = lax.axis_index("c")
        sid = lax.axis_index("s")
        wid = sid * _NC + cid
        base = wid * EW

        def do_table(idx_hbm, col, do_ns):
            pltpu.sync_copy(idx_hbm.at[pl.ds(base, EW)], idx_v)
            for s in range(NSUP):
                off = s * SUP

                def fire(cc, carry):
                    pltpu.async_copy(
                        hn_hbm.at[idx_v.at[pl.ds(off + cc * CH, CH)]],
                        rows_v.at[pl.ds(cc * CH, CH)], gsem)
                    if do_ns:
                        pltpu.async_copy(
                            ns_hbm.at[idx_v.at[pl.ds(off + cc * CH, CH)]],
                            nsout_v.at[pl.ds(off + cc * CH, CH)], gsem)
                    return carry

                def drain(cc, carry):
                    pltpu.make_async_copy(
                        hn_hbm.at[idx_v.at[pl.ds(off + cc * CH, CH)]],
                        rows_v.at[pl.ds(cc * CH, CH)], gsem).wait()
                    if do_ns:
                        pltpu.make_async_copy(
                            ns_hbm.at[idx_v.at[pl.ds(off + cc * CH, CH)]],
                            nsout_v.at[pl.ds(off + cc * CH, CH)], gsem).wait()
                    return carry

                lax.fori_loop(0, NCH, fire, 0)
                lax.fori_loop(0, NCH, drain, 0)
                pltpu.sync_copy(
                    rows_v,
                    hnij_hbm.at[pl.ds(base + off, SUP), pl.ds(col, DS)])
            if do_ns:
                pltpu.sync_copy(nsout_v, nsi_hbm.at[pl.ds(base, EW)])

        do_table(ii_hbm, 0, True)
        do_table(jj_hbm, DS, False)

    return sc_gather


def _edge_body(rel_ref, hnij_ref, lq_ref, rq_ref,
               wproj_ref, bproj_ref, wl_ref, wr_ref, wc_ref, bc_ref,
               logit_ref):
    ds = wproj_ref.shape[1]
    wl = wl_ref[...]
    wr = wr_ref[...]
    z = jnp.zeros((ds, ds), _f32)
    wd = jnp.concatenate(
        [jnp.concatenate([wl[0:ds], z], axis=1),
         jnp.concatenate([z, wr[0:ds]], axis=1)], axis=0)
    wrelb = jnp.concatenate([wl[ds:2 * ds], wr[ds:2 * ds]], axis=1)
    wc2 = jnp.concatenate([z, wc_ref[...]], axis=0)
    lqrq = jnp.concatenate([lq_ref[0], rq_ref[0]], axis=1)
    relp = _dot(rel_ref[0], wproj_ref[...]) + bproj_ref[...]
    alar = _leaky(_dot(hnij_ref[0], wd) + _dot(relp, wrelb) + lqrq)
    m2 = _dot(alar, wc2) + bc_ref[...]
    al = alar[:, 0:ds]
    logit_ref[0] = jnp.sum(al * m2, axis=1, keepdims=True)


def _edge_call(B, EQ, D, DS, GIN):
    return pl.pallas_call(
        _edge_body,
        grid=(B,),
        in_specs=[
            pl.BlockSpec((1, EQ, D), lambda b: (b, 0, 0)),
            pl.BlockSpec((1, EQ, 2 * DS), lambda b: (b, 0, 0)),
            pl.BlockSpec((1, 1, DS), lambda b: (b, 0, 0)),
            pl.BlockSpec((1, 1, DS), lambda b: (b, 0, 0)),
            pl.BlockSpec((D, DS), lambda b: (0, 0)),
            pl.BlockSpec((1, DS), lambda b: (0, 0)),
            pl.BlockSpec((GIN, DS), lambda b: (0, 0)),
            pl.BlockSpec((GIN, DS), lambda b: (0, 0)),
            pl.BlockSpec((DS, DS), lambda b: (0, 0)),
            pl.BlockSpec((1, DS), lambda b: (0, 0)),
        ],
        out_specs=pl.BlockSpec((1, EQ, 1), lambda b: (b, 0, 0)),
        out_shape=jax.ShapeDtypeStruct((B, EQ, 1), _f32),
    )


def _topk_body(k_ref, lg_ref, ns_ref, out_ref):
    lg = lg_ref[...]
    m = jnp.max(lg, axis=1, keepdims=True)
    p = jnp.exp(lg - m)
    z = jnp.sum(p, axis=1, keepdims=True)
    att = (p / z) * ns_ref[...]
    bits = lax.bitcast_convert_type(att, jnp.int32)
    k = k_ref[0]
    b = att.shape[0]

    def body(_, carry):
        lo, hi = carry
        mid = lo + lax.shift_right_logical(hi - lo, 1)
        cnt = jnp.sum((bits >= mid).astype(jnp.int32), axis=1, keepdims=True)
        ge = cnt >= k
        return jnp.where(ge, mid, lo), jnp.where(ge, hi, mid)

    lo0 = jnp.zeros((b, 1), jnp.int32)
    hi0 = jnp.full((b, 1), jnp.int32(0x7FFFFFFF))
    lo, _ = lax.fori_loop(0, 31, body, (lo0, hi0))
    out_ref[...] = jnp.where(bits >= lo, att, 0.0)


def _topk_call(B, EQ):
    return pl.pallas_call(
        _topk_body,
        in_specs=[
            pl.BlockSpec(memory_space=pltpu.SMEM),
            pl.BlockSpec((B, EQ), lambda: (0, 0)),
            pl.BlockSpec((B, EQ), lambda: (0, 0)),
        ],
        out_specs=pl.BlockSpec((B, EQ), lambda: (0, 0)),
        out_shape=jax.ShapeDtypeStruct((B, EQ), _f32),
    )


def _sc_scatter_call(N, JC):
    mesh = plsc.VectorSubcoreMesh(core_axis_name="c", subcore_axis_name="s")

    @functools.partial(
        pl.kernel,
        out_type=jax.ShapeDtypeStruct((_NC, N), _f32),
        mesh=mesh,
        scratch_types=[
            pltpu.VMEM((JC, 128), _f32),
            pltpu.VMEM((JC, 128), jnp.int32),
            pltpu.VMEM((N,), _f32),
            pltpu.VMEM_SHARED((N,), _f32),
            pltpu.SemaphoreType.DMA,
        ],
    )
    def sc_scatter(val_hbm, idx_hbm, out_hbm, val_v, idx_v, z_v, acc_sh, sem):
        cid = lax.axis_index("c")
        sid = lax.axis_index("s")
        wid = sid * _NC + cid

        @pl.when(sid == 0)
        def _zero():
            def zb(t, carry):
                z_v[pl.ds(t * 16, 16)] = jnp.zeros((16,), _f32)
                return carry
            lax.fori_loop(0, N // 16, zb, 0)
            pltpu.sync_copy(z_v, acc_sh)

        pltpu.sync_copy(val_hbm.at[wid], val_v)
        pltpu.sync_copy(idx_hbm.at[wid], idx_v)
        plsc.subcore_barrier()

        def scat(j, carry):
            pltpu.sync_copy(val_v.at[j], acc_sh.at[idx_v.at[j]], add=True)
            return carry
        lax.fori_loop(0, JC, scat, 0)

        plsc.subcore_barrier()

        @pl.when(sid == 0)
        def _out():
            pltpu.sync_copy(acc_sh, out_hbm.at[cid])

    return sc_scatter


def _add_body(p_ref, o_ref):
    o_ref[...] = p_ref[0, :] + p_ref[1, :]


def _add_call(N):
    return pl.pallas_call(
        _add_body,
        out_shape=jax.ShapeDtypeStruct((N,), _f32),
    )


def kernel(node_score, selected_edges_l, visited_node_representation,
           rel_emb_l, query_src_ts_emb, query_rel_emb, W_proj, b_proj,
           W_ps, b_ps, Wl, bl, Wr, br, Wc, bc, max_edges):
    B = query_rel_emb.shape[0]
    E = selected_edges_l.shape[0]
    EQ = E // B
    N = node_score.shape[0]
    D = visited_node_representation.shape[1]
    DS = W_proj.shape[1]
    SDS = W_ps.shape[1]

    GIN = Wl.shape[0]

    hn, lq, rq = _prep_call(N, B, D, DS, SDS)(
        visited_node_representation, W_proj, b_proj.reshape(1, DS),
        W_ps, b_ps.reshape(1, SDS), Wl, bl.reshape(1, DS),
        Wr, br.reshape(1, DS), query_src_ts_emb, query_rel_emb)

    idx_i = selected_edges_l[:, 6]
    idx_j = selected_edges_l[:, 7]

    hnij, nsi = _sc_gather_call(N, E, DS)(
        hn, node_score, idx_i, idx_j)

    logits = _edge_call(B, EQ, D, DS, GIN)(
        rel_emb_l.reshape(B, EQ, D), hnij.reshape(B, EQ, 2 * DS),
        lq.reshape(B, 1, DS), rq.reshape(B, 1, DS), W_proj,
        b_proj.reshape(1, DS), Wl, Wr, Wc, bc.reshape(1, DS))

    pruned = _topk_call(B, EQ)(
        jnp.asarray(max_edges, jnp.int32).reshape(1),
        logits.reshape(B, EQ), nsi.reshape(B, EQ))

    JC = -(-(E // _NW) // 128)
    EPAD = _NW * JC * 128
    pv = jnp.concatenate([pruned.reshape(E),
                          jnp.zeros((EPAD - E,), _f32)])
    pj = jnp.concatenate([idx_j, jnp.zeros((EPAD - E,), jnp.int32)])

    partials = _sc_scatter_call(N, JC)(
        pv.reshape(_NW, JC, 128), pj.reshape(_NW, JC, 128))

    return _add_call(N)(partials)
